# baseline (speedup 1.0000x reference)
import dataclasses
import functools

import jax
import jax.numpy as jnp
from jax import lax
from jax.experimental import pallas as pl
from jax.experimental.pallas import tpu as pltpu
from jax.experimental.pallas import tpu_sc as plsc

LANES = 16
STREAM_IDX = 128


def _tree_sum(vals):
  vals = list(vals)
  while len(vals) > 1:
    nxt = [a + b for a, b in zip(vals[0::2], vals[1::2])]
    if len(vals) % 2:
      nxt.append(vals[-1])
    vals = nxt
  return vals[0]


def _make_sc_call(B, CTX, NEG, D, dtype):
  mesh = plsc.VectorSubcoreMesh(core_axis_name="c", subcore_axis_name="s")
  NC = mesh.num_cores
  NW = NC * mesh.num_subcores
  assert B % (NW * LANES) == 0
  PER_W = B // NW
  C = LANES
  NCHUNK = PER_W // C
  D2 = 2 * D

  def body(ctx_idx_hbm, tgt_idx_hbm, neg_idx_hbm, emb_in_hbm, emb_out_hbm,
           pos_hbm, neg_hbm,
           ctx_idx_v, tgt_idx_v, neg_idx_v,
           ctx_row_v, tgt_row_v, neg_row_v,
           ctx_rows, tgt_rows, neg_rows,
           mean_v, pos_buf, neg_buf, sem):
    cid = lax.axis_index("c")
    sid = lax.axis_index("s")
    wid = sid * NC + cid
    base = wid * PER_W

    pltpu.sync_copy(ctx_idx_hbm.at[pl.ds(base * CTX, PER_W * CTX)], ctx_idx_v)
    pltpu.sync_copy(tgt_idx_hbm.at[pl.ds(base, PER_W)], tgt_idx_v)
    pltpu.sync_copy(neg_idx_hbm.at[pl.ds(base * NEG, PER_W * NEG)], neg_idx_v)

    @pl.loop(0, PER_W * CTX, step=LANES)
    def _shift_ctx(i):
      ctx_row_v[pl.ds(i, LANES)] = lax.shift_right_logical(
          ctx_idx_v[pl.ds(i, LANES)], 1)

    @pl.loop(0, PER_W, step=LANES)
    def _shift_tgt(i):
      tgt_row_v[pl.ds(i, LANES)] = lax.shift_right_logical(
          tgt_idx_v[pl.ds(i, LANES)], 1)

    @pl.loop(0, PER_W * NEG, step=LANES)
    def _shift_neg(i):
      neg_row_v[pl.ds(i, LANES)] = lax.shift_right_logical(
          neg_idx_v[pl.ds(i, LANES)], 1)

    e_iota = lax.iota(jnp.int32, LANES)

    def stream_pieces(n):
      pieces, o = [], 0
      while o < n:
        w = min(STREAM_IDX, n - o)
        pieces.append((o, w))
        o += w
      return pieces

    def issue(ci):
      descs = []
      co = ci * C
      for o, w in stream_pieces(C * CTX):
        descs.append(pltpu.make_async_copy(
            emb_in_hbm.at[ctx_row_v.at[pl.ds(co * CTX + o, w)]],
            ctx_rows.at[pl.ds(o, w)], sem))
      for o, w in stream_pieces(C * NEG):
        descs.append(pltpu.make_async_copy(
            emb_out_hbm.at[neg_row_v.at[pl.ds(co * NEG + o, w)]],
            neg_rows.at[pl.ds(o, w)], sem))
      descs.append(pltpu.make_async_copy(
          emb_out_hbm.at[tgt_row_v.at[pl.ds(co, C)]],
          tgt_rows, sem))
      for d_ in descs:
        d_.start()
      return descs

    row_ctx = [e_iota * CTX + j for j in range(CTX)]
    row_neg = [e_iota * NEG + n for n in range(NEG)]
    half = jnp.full((LANES,), D, jnp.int32)
    one = jnp.full((LANES,), 1, jnp.int32)

    def parity(idx_ref, rows):
      return [(plsc.load_gather(idx_ref, [r]) & one) * half for r in rows]

    def compute(ci):
      co = ci * C
      abs_ctx = [co * CTX + r for r in row_ctx]
      abs_neg = [co * NEG + r for r in row_neg]
      par_ctx = parity(ctx_idx_v, abs_ctx)
      par_tgt = (plsc.load_gather(tgt_idx_v, [co + e_iota]) & one) * half

      @pl.loop(0, D)
      def _mean(d):
        cold = (e_iota + d) & (D - 1)
        m = _tree_sum([plsc.load_gather(ctx_rows, [row_ctx[j],
                                                   par_ctx[j] + cold])
                       for j in range(CTX)])
        mean_v[pl.ds(d * LANES, LANES)] = m * (1.0 / CTX)

      par_neg = parity(neg_idx_v, abs_neg)

      def dbody(d, carry):
        pos_acc, neg_accs = carry
        cold = (e_iota + d) & (D - 1)
        m = mean_v[pl.ds(d * LANES, LANES)]
        pos_acc = pos_acc + plsc.load_gather(
            tgt_rows, [e_iota, par_tgt + cold]) * m
        neg_accs = tuple(
            neg_accs[n] + plsc.load_gather(
                neg_rows, [row_neg[n], par_neg[n] + cold]) * m
            for n in range(NEG))
        return pos_acc, neg_accs

      zero = jnp.zeros((LANES,), jnp.float32)
      pos_acc, neg_accs = lax.fori_loop(0, D, dbody, (zero, (zero,) * NEG))

      pos_buf[pl.ds(0, LANES)] = pos_acc
      for n in range(NEG):
        plsc.store_scatter(neg_buf,
                           [e_iota * NEG + n],
                           neg_accs[n])
      pltpu.sync_copy(pos_buf, pos_hbm.at[pl.ds(base + co, C)])
      pltpu.sync_copy(neg_buf, neg_hbm.at[pl.ds((base + co) * NEG, C * NEG)])

    @pl.loop(0, NCHUNK)
    def _chunk(ci):
      descs = issue(ci)
      for d_ in descs:
        d_.wait()
      compute(ci)

  cp = pltpu.CompilerParams()
  fields = getattr(pltpu.CompilerParams, "__dataclass_fields__", {})
  if "needs_layout_passes" in fields:
    cp = dataclasses.replace(cp, needs_layout_passes=False)

  return pl.kernel(
      body,
      out_type=(jax.ShapeDtypeStruct((B,), dtype),
                jax.ShapeDtypeStruct((B * NEG,), dtype)),
      mesh=mesh,
      compiler_params=cp,
      scratch_types=[
          pltpu.VMEM((PER_W * CTX,), jnp.int32),
          pltpu.VMEM((PER_W,), jnp.int32),
          pltpu.VMEM((PER_W * NEG,), jnp.int32),
          pltpu.VMEM((PER_W * CTX,), jnp.int32),
          pltpu.VMEM((PER_W,), jnp.int32),
          pltpu.VMEM((PER_W * NEG,), jnp.int32),
          pltpu.VMEM((C * CTX, D2), dtype),
          pltpu.VMEM((C, D2), dtype),
          pltpu.VMEM((C * NEG, D2), dtype),
          pltpu.VMEM((D * LANES,), dtype),
          pltpu.VMEM((LANES,), dtype),
          pltpu.VMEM((C * NEG,), dtype),
          pltpu.SemaphoreType.DMA,
      ],
  )


def kernel(input_ids, labels, negative_samples, emb_in, emb_out):
  B, CTX = input_ids.shape
  NEG = negative_samples.shape[1]
  V, D = emb_in.shape
  ctx_idx = input_ids.reshape(-1).astype(jnp.int32)
  tgt_idx = labels.reshape(-1).astype(jnp.int32)
  neg_idx = negative_samples.reshape(-1).astype(jnp.int32)
  emb_in2 = emb_in.reshape(V // 2, 2 * D)
  emb_out2 = emb_out.reshape(V // 2, 2 * D)
  call = _make_sc_call(B, CTX, NEG, D, emb_in.dtype)
  pos, neg = call(ctx_idx, tgt_idx, neg_idx, emb_in2, emb_out2)
  return pos, neg.reshape(B, NEG)


# speedup vs baseline: 1.3674x; 1.3674x over previous
import dataclasses
import functools

import jax
import jax.numpy as jnp
from jax import lax
from jax.experimental import pallas as pl
from jax.experimental.pallas import tpu as pltpu
from jax.experimental.pallas import tpu_sc as plsc

LANES = 16


def _tree_sum(vals):
  vals = list(vals)
  while len(vals) > 1:
    nxt = [a + b for a, b in zip(vals[0::2], vals[1::2])]
    if len(vals) % 2:
      nxt.append(vals[-1])
    vals = nxt
  return vals[0]


def _make_sc_call(B, CTX, NEG, D, dtype):
  mesh = plsc.VectorSubcoreMesh(core_axis_name="c", subcore_axis_name="s")
  NC = mesh.num_cores
  NW = NC * mesh.num_subcores
  assert B % (NW * LANES) == 0
  PER_W = B // NW
  C = LANES
  NCHUNK = PER_W // C
  assert NCHUNK % 2 == 0

  def body(ctx_idx_hbm, tgt_idx_hbm, neg_idx_hbm, emb_in_hbm, emb_out_hbm,
           pos_hbm, neg_hbm,
           ctx_idx_v, tgt_idx_v, neg_idx_v,
           ctx_rows, tgt_rows, neg_rows,
           mean_v, pos_buf, neg_buf, sems):
    cid = lax.axis_index("c")
    sid = lax.axis_index("s")
    wid = sid * NC + cid
    base = wid * PER_W

    pltpu.sync_copy(ctx_idx_hbm.at[pl.ds(base * CTX, PER_W * CTX)], ctx_idx_v)
    pltpu.sync_copy(tgt_idx_hbm.at[pl.ds(base, PER_W)], tgt_idx_v)
    pltpu.sync_copy(neg_idx_hbm.at[pl.ds(base * NEG, PER_W * NEG)], neg_idx_v)

    e_iota = lax.iota(jnp.int32, LANES)

    WAVE = 4 * LANES

    def row_waves(ci, b):
      co = ci * C
      waves = []
      for tbl, idx_ref, off, rows_b, n in (
          (emb_in_hbm, ctx_idx_v, co * CTX, ctx_rows.at[b], C * CTX),
          (emb_out_hbm, neg_idx_v, co * NEG, neg_rows.at[b], C * NEG),
          (emb_out_hbm, tgt_idx_v, co, tgt_rows.at[b], C)):
        for w0 in range(0, n, WAVE):
          waves.append((tbl, idx_ref, off, rows_b, w0, min(WAVE, n - w0)))
      return waves

    def fire(wave, sem):
      tbl, idx_ref, off, rows_b, w0, n = wave
      for g in range(0, n, LANES):
        vec = idx_ref[pl.ds(off + w0 + g, LANES)]
        for l in range(LANES):
          v = lax.squeeze(lax.slice(vec, (l,), (l + 1,)), (0,))
          pltpu.make_async_copy(tbl.at[pl.ds(v, 1)],
                                rows_b.at[pl.ds(w0 + g + l, 1)], sem).start()

    def drain_wave(wave, sem):
      tbl, idx_ref, off, rows_b, w0, n = wave
      pltpu.make_async_copy(tbl.at[pl.ds(0, n)],
                            rows_b.at[pl.ds(w0, n)], sem).wait()

    def issue(ci, b, ahead):
      waves = row_waves(ci, b)
      for i, w in enumerate(waves):
        fire(w, sems.at[b])
        if i >= ahead:
          drain_wave(waves[i - ahead], sems.at[b])
      return waves[len(waves) - ahead:]

    def drain(tail, b):
      for w in tail:
        drain_wave(w, sems.at[b])

    row_ctx = [e_iota * CTX + j for j in range(CTX)]
    row_neg = [e_iota * NEG + n for n in range(NEG)]

    def compute(ci, b):
      co = ci * C
      crows = ctx_rows.at[b]
      nrows = neg_rows.at[b]
      trows = tgt_rows.at[b]

      @pl.loop(0, D)
      def _mean(d):
        cold = (e_iota + d) & (D - 1)
        m = _tree_sum([plsc.load_gather(crows, [row_ctx[j], cold])
                       for j in range(CTX)])
        mean_v[pl.ds(d * LANES, LANES)] = m * (1.0 / CTX)

      def dbody(d, carry):
        pos_acc, neg_accs = carry
        cold = (e_iota + d) & (D - 1)
        m = mean_v[pl.ds(d * LANES, LANES)]
        pos_acc = pos_acc + plsc.load_gather(trows, [e_iota, cold]) * m
        neg_accs = tuple(
            neg_accs[n] + plsc.load_gather(nrows, [row_neg[n], cold]) * m
            for n in range(NEG))
        return pos_acc, neg_accs

      zero = jnp.zeros((LANES,), jnp.float32)
      pos_acc, neg_accs = lax.fori_loop(0, D, dbody, (zero, (zero,) * NEG))

      pos_buf[...] = pos_acc
      for n in range(NEG):
        plsc.store_scatter(neg_buf, [e_iota * NEG + n], neg_accs[n])
      pltpu.sync_copy(pos_buf, pos_hbm.at[pl.ds(base + co, C)])
      pltpu.sync_copy(neg_buf, neg_hbm.at[pl.ds((base + co) * NEG, C * NEG)])

    @pl.loop(0, NCHUNK)
    def _chunk(ci):
      tail = issue(ci, 0, ahead=2)
      drain(tail, 0)
      compute(ci, 0)

  cp = pltpu.CompilerParams()
  fields = getattr(pltpu.CompilerParams, "__dataclass_fields__", {})
  if "needs_layout_passes" in fields:
    cp = dataclasses.replace(cp, needs_layout_passes=False)

  return pl.kernel(
      body,
      out_type=(jax.ShapeDtypeStruct((B,), dtype),
                jax.ShapeDtypeStruct((B * NEG,), dtype)),
      mesh=mesh,
      compiler_params=cp,
      scratch_types=[
          pltpu.VMEM((PER_W * CTX,), jnp.int32),
          pltpu.VMEM((PER_W,), jnp.int32),
          pltpu.VMEM((PER_W * NEG,), jnp.int32),
          pltpu.VMEM((1, C * CTX, D), dtype),
          pltpu.VMEM((1, C, D), dtype),
          pltpu.VMEM((1, C * NEG, D), dtype),
          pltpu.VMEM((D * LANES,), dtype),
          pltpu.VMEM((LANES,), dtype),
          pltpu.VMEM((C * NEG,), dtype),
          pltpu.SemaphoreType.DMA((2,)),
      ],
  )


def kernel(input_ids, labels, negative_samples, emb_in, emb_out):
  B, CTX = input_ids.shape
  NEG = negative_samples.shape[1]
  V, D = emb_in.shape
  ctx_idx = input_ids.reshape(-1).astype(jnp.int32)
  tgt_idx = labels.reshape(-1).astype(jnp.int32)
  neg_idx = negative_samples.reshape(-1).astype(jnp.int32)
  call = _make_sc_call(B, CTX, NEG, D, emb_in.dtype)
  pos, neg = call(ctx_idx, tgt_idx, neg_idx, emb_in, emb_out)
  return pos, neg.reshape(B, NEG)
